# baseline (speedup 1.0000x reference)
import jax
import jax.numpy as jnp
from jax import lax
from jax.experimental import pallas as pl
from jax.experimental.pallas import tpu as pltpu
from jax.experimental.pallas import tpu_sc as plsc

F = 26
V = 100000
D = 32
B = 4096
FIELD_DIM = 3846

NC = 2
NS = 16
NW = NC * NS
SPW = B // NW
NSLOT = 4

NPAIR = (F * (F - 1)) // 2
NQ = (F * D + 127) // 128
NROW = F * NQ
IDX_W = 224
FFM_OFF = 32
CHUNKS = [(FFM_OFF, 128), (FFM_OFF + 128, NROW - 128)]

VB = 256
VPAD = 100096
NVB = VPAD // VB

_GDN = lax.GatherDimensionNumbers(
    offset_dims=(), collapsed_slice_dims=(0,), start_index_map=(0,))


def _permute(v, idx):
    return lax.gather(v, idx[:, None], _GDN, (1,),
                      mode=lax.GatherScatterMode.PROMISE_IN_BOUNDS)


def _hsum(v, lanes):
    for sh in (8, 4, 2, 1):
        v = v + _permute(v, lanes ^ sh)
    return v


def _tr_body(t_ref, o_ref):
    x = t_ref[...]
    x = x.reshape(128, VB)
    ident = jnp.eye(128, dtype=jnp.float32)
    o_ref[...] = lax.dot_general(
        x, ident, (((0,), (0,)), ((), ())),
        preferred_element_type=jnp.float32)


@jax.jit
def _relayout_tc(tt):
    return pl.pallas_call(
        _tr_body,
        grid=(NQ, NVB),
        in_specs=[pl.BlockSpec((4, D, VB), lambda q, k: (q, 0, k))],
        out_specs=pl.BlockSpec((VB, 128), lambda q, k: (q * NVB + k, 0)),
        out_shape=jax.ShapeDtypeStruct((NQ * VPAD, 128), jnp.float32),
    )(tt)


def _sc_body(i_hbm, lin_hbm, pt_hbm, bias_hbm, out_hbm,
             idx_v, g_v, l_v, outb_v, bias_v, sems, isems):
    wid = lax.axis_index("s") * NC + lax.axis_index("c")
    base = wid * SPW

    pltpu.sync_copy(bias_hbm, bias_v)

    lanes = lax.iota(jnp.int32, 16)
    zero_f = jnp.zeros((16,), jnp.float32)
    bias_vec = bias_v[...]
    for s in range(NSLOT):
        l_v[s][pl.ds(16, 16)] = zero_f

    def idx_start(g, s):
        pltpu.async_copy(i_hbm.at[base + g], idx_v[s], isems[s])

    def idx_wait(s):
        pltpu.make_async_copy(i_hbm.at[base], idx_v[s], isems[s]).wait()

    def issue(s):
        pltpu.async_copy(lin_hbm.at[idx_v[s].at[pl.ds(0, F)]],
                         l_v[s].at[pl.ds(0, F)], sems[s])
        for off, n in CHUNKS:
            pltpu.async_copy(pt_hbm.at[idx_v[s].at[pl.ds(off, n)]],
                             g_v[s].at[pl.ds(off - FFM_OFF, n)], sems[s])

    def drain(s):
        pltpu.make_async_copy(lin_hbm.at[idx_v[s].at[pl.ds(0, F)]],
                              l_v[s].at[pl.ds(0, F)], sems[s]).wait()
        for off, n in CHUNKS:
            pltpu.make_async_copy(pt_hbm.at[idx_v[s].at[pl.ds(off, n)]],
                                  g_v[s].at[pl.ds(off - FFM_OFF, n)],
                                  sems[s]).wait()

    def compute(s):
        gs = g_v[s]

        def pbody(_, carry):
            i, j, a0, a1 = carry
            ra = i * NQ + (j >> 2)
            ca = (j & 3) * 32
            rb = j * NQ + (i >> 2)
            cb = (i & 3) * 32
            a0 = a0 + gs[ra, pl.ds(ca, 16)] * gs[rb, pl.ds(cb, 16)]
            a1 = a1 + gs[ra, pl.ds(ca + 16, 16)] * gs[rb, pl.ds(cb + 16, 16)]
            j2 = j + 1
            wrap = j2 == F
            i2 = jnp.where(wrap, i + 1, i)
            j3 = jnp.where(wrap, i + 2, j2)
            return i2, j3, a0, a1

        _, _, a0, a1 = lax.fori_loop(
            0, NPAIR, pbody,
            (jnp.int32(0), jnp.int32(1), zero_f, zero_f), unroll=13)
        lin = l_v[s][pl.ds(0, 16)] + l_v[s][pl.ds(16, 16)]
        return _hsum(a0 + a1 + lin, lanes) + bias_vec

    for s in range(NSLOT):
        idx_start(s, s)
    for s in range(2):
        idx_wait(s)
        issue(s)

    def lbody(t, res):
        g0 = t * NSLOT
        for s in range(NSLOT):
            g = g0 + s
            drain(s)

            @pl.when(g + 2 < SPW)
            def _():
                idx_wait((s + 2) % NSLOT)
                issue((s + 2) % NSLOT)

            @pl.when(g + NSLOT < SPW)
            def _():
                idx_start(g + NSLOT, s)

            res = jnp.where(lanes == g % 16, compute(s), res)

            @pl.when(g % 16 == 15)
            def _():
                outb_v[pl.ds(g - 15, 16)] = res
        return res

    lax.fori_loop(0, SPW // NSLOT, lbody, zero_f)

    for k in range(SPW // 16):
        v = outb_v[pl.ds(k * 16, 16)]
        outb_v[pl.ds(k * 16, 16)] = 1.0 / (1.0 + jnp.exp(-v))
    pltpu.sync_copy(outb_v, out_hbm.at[pl.ds(base, SPW)])


@jax.jit
def _ffm_sc(i_rows, lin_table, pt, bias16):
    mesh = plsc.VectorSubcoreMesh(core_axis_name="c", subcore_axis_name="s")
    run = pl.kernel(
        _sc_body,
        out_type=jax.ShapeDtypeStruct((B,), jnp.float32),
        mesh=mesh,
        compiler_params=pltpu.CompilerParams(use_tc_tiling_on_sc=False),
        scratch_types=[
            [pltpu.VMEM((IDX_W,), jnp.int32) for _ in range(NSLOT)],
            [pltpu.VMEM((NROW, 128), jnp.float32) for _ in range(NSLOT)],
            [pltpu.VMEM((32,), jnp.float32) for _ in range(NSLOT)],
            pltpu.VMEM((SPW,), jnp.float32),
            pltpu.VMEM((16,), jnp.float32),
            [pltpu.SemaphoreType.DMA for _ in range(NSLOT)],
            [pltpu.SemaphoreType.DMA for _ in range(NSLOT)],
        ],
    )
    return run(i_rows, lin_table, pt, bias16)


def kernel(x, offsets, lin_table, lin_bias, ffm_tables):
    idx = x + offsets[None, :]
    tt = jnp.transpose(ffm_tables, (0, 2, 1))
    pt = _relayout_tc(tt)
    ffm_idx = (idx[:, :, None]
               + (jnp.arange(NQ, dtype=jnp.int32) * VPAD)[None, None, :])
    i_rows = jnp.concatenate(
        [idx, jnp.zeros((B, FFM_OFF - F), jnp.int32),
         ffm_idx.reshape(B, NROW),
         jnp.zeros((B, IDX_W - FFM_OFF - NROW), jnp.int32)],
        axis=1)
    bias16 = jnp.broadcast_to(lin_bias, (16,)).astype(jnp.float32)
    return _ffm_sc(i_rows, lin_table.reshape(V), pt, bias16)


# speedup vs baseline: 2.4759x; 2.4759x over previous
import jax
import jax.numpy as jnp
from jax import lax
from jax.experimental import pallas as pl
from jax.experimental.pallas import tpu as pltpu
from jax.experimental.pallas import tpu_sc as plsc

F = 26
V = 100000
D = 32
B = 4096
FIELD_DIM = 3846

NC = 2
NS = 16
NW = NC * NS
SPW = B // NW
NSLOT = 4

NPAIR = (F * (F - 1)) // 2
NQ = (F * D + 127) // 128
NROW = F * NQ
IDX_W = 224
FFM_OFF = 32
CHUNKS = [(FFM_OFF, 128), (FFM_OFF + 128, NROW - 128)]

VB = 1024
VPAD = 100352
NVB = VPAD // VB

_GDN = lax.GatherDimensionNumbers(
    offset_dims=(), collapsed_slice_dims=(0,), start_index_map=(0,))


def _permute(v, idx):
    return lax.gather(v, idx[:, None], _GDN, (1,),
                      mode=lax.GatherScatterMode.PROMISE_IN_BOUNDS)


def _hsum(v, lanes):
    for sh in (8, 4, 2, 1):
        v = v + _permute(v, lanes ^ sh)
    return v


def _tr_body(t_ref, o_ref):
    x = t_ref[...]
    x = x.reshape(128, VB)
    o_ref[...] = jnp.transpose(x)


@jax.jit
def _relayout_tc(tt):
    return pl.pallas_call(
        _tr_body,
        grid=(NQ, NVB),
        in_specs=[pl.BlockSpec((4, D, VB), lambda q, k: (q, 0, k))],
        out_specs=pl.BlockSpec((VB, 128), lambda q, k: (q * NVB + k, 0)),
        out_shape=jax.ShapeDtypeStruct((NQ * VPAD, 128), jnp.float32),
    )(tt)


def _sc_body(i_hbm, lin_hbm, pt_hbm, bias_hbm, out_hbm,
             idx_v, g_v, l_v, outb_v, bias_v, sems, isems):
    wid = lax.axis_index("s") * NC + lax.axis_index("c")
    base = wid * SPW

    pltpu.sync_copy(bias_hbm, bias_v)

    lanes = lax.iota(jnp.int32, 16)
    zero_f = jnp.zeros((16,), jnp.float32)
    bias_vec = bias_v[...]
    for s in range(NSLOT):
        l_v[s][pl.ds(16, 16)] = zero_f

    def idx_start(g, s):
        pltpu.async_copy(i_hbm.at[base + g], idx_v[s], isems[s])

    def idx_wait(s):
        pltpu.make_async_copy(i_hbm.at[base], idx_v[s], isems[s]).wait()

    def issue(s):
        pltpu.async_copy(lin_hbm.at[idx_v[s].at[pl.ds(0, F)]],
                         l_v[s].at[pl.ds(0, F)], sems[s])
        for off, n in CHUNKS:
            pltpu.async_copy(pt_hbm.at[idx_v[s].at[pl.ds(off, n)]],
                             g_v[s].at[pl.ds(off - FFM_OFF, n)], sems[s])

    def drain(s):
        pltpu.make_async_copy(lin_hbm.at[idx_v[s].at[pl.ds(0, F)]],
                              l_v[s].at[pl.ds(0, F)], sems[s]).wait()
        for off, n in CHUNKS:
            pltpu.make_async_copy(pt_hbm.at[idx_v[s].at[pl.ds(off, n)]],
                                  g_v[s].at[pl.ds(off - FFM_OFF, n)],
                                  sems[s]).wait()

    def compute(s):
        gs = g_v[s]

        def pbody(_, carry):
            i, j, a0, a1 = carry
            ra = i * NQ + (j >> 2)
            ca = (j & 3) * 32
            rb = j * NQ + (i >> 2)
            cb = (i & 3) * 32
            a0 = a0 + gs[ra, pl.ds(ca, 16)] * gs[rb, pl.ds(cb, 16)]
            a1 = a1 + gs[ra, pl.ds(ca + 16, 16)] * gs[rb, pl.ds(cb + 16, 16)]
            j2 = j + 1
            wrap = j2 == F
            i2 = jnp.where(wrap, i + 1, i)
            j3 = jnp.where(wrap, i + 2, j2)
            return i2, j3, a0, a1

        _, _, a0, a1 = lax.fori_loop(
            0, NPAIR, pbody,
            (jnp.int32(0), jnp.int32(1), zero_f, zero_f), unroll=13)
        lin = l_v[s][pl.ds(0, 16)] + l_v[s][pl.ds(16, 16)]
        return _hsum(a0 + a1 + lin, lanes) + bias_vec

    for s in range(NSLOT):
        idx_start(s, s)
    for s in range(2):
        idx_wait(s)
        issue(s)

    def lbody(t, res):
        g0 = t * NSLOT
        for s in range(NSLOT):
            g = g0 + s
            drain(s)

            @pl.when(g + 2 < SPW)
            def _():
                idx_wait((s + 2) % NSLOT)
                issue((s + 2) % NSLOT)

            @pl.when(g + NSLOT < SPW)
            def _():
                idx_start(g + NSLOT, s)

            res = jnp.where(lanes == g % 16, compute(s), res)

            @pl.when(g % 16 == 15)
            def _():
                outb_v[pl.ds(g - 15, 16)] = res
        return res

    lax.fori_loop(0, SPW // NSLOT, lbody, zero_f)

    for k in range(SPW // 16):
        v = outb_v[pl.ds(k * 16, 16)]
        outb_v[pl.ds(k * 16, 16)] = 1.0 / (1.0 + jnp.exp(-v))
    pltpu.sync_copy(outb_v, out_hbm.at[pl.ds(base, SPW)])


@jax.jit
def _ffm_sc(i_rows, lin_table, pt, bias16):
    mesh = plsc.VectorSubcoreMesh(core_axis_name="c", subcore_axis_name="s")
    run = pl.kernel(
        _sc_body,
        out_type=jax.ShapeDtypeStruct((B,), jnp.float32),
        mesh=mesh,
        compiler_params=pltpu.CompilerParams(use_tc_tiling_on_sc=False),
        scratch_types=[
            [pltpu.VMEM((IDX_W,), jnp.int32) for _ in range(NSLOT)],
            [pltpu.VMEM((NROW, 128), jnp.float32) for _ in range(NSLOT)],
            [pltpu.VMEM((32,), jnp.float32) for _ in range(NSLOT)],
            pltpu.VMEM((SPW,), jnp.float32),
            pltpu.VMEM((16,), jnp.float32),
            [pltpu.SemaphoreType.DMA for _ in range(NSLOT)],
            [pltpu.SemaphoreType.DMA for _ in range(NSLOT)],
        ],
    )
    return run(i_rows, lin_table, pt, bias16)


def kernel(x, offsets, lin_table, lin_bias, ffm_tables):
    idx = x + offsets[None, :]
    tt = jnp.transpose(ffm_tables, (0, 2, 1))
    pt = _relayout_tc(tt)
    ffm_idx = (idx[:, :, None]
               + (jnp.arange(NQ, dtype=jnp.int32) * VPAD)[None, None, :])
    i_rows = jnp.concatenate(
        [idx, jnp.zeros((B, FFM_OFF - F), jnp.int32),
         ffm_idx.reshape(B, NROW),
         jnp.zeros((B, IDX_W - FFM_OFF - NROW), jnp.int32)],
        axis=1)
    bias16 = jnp.broadcast_to(lin_bias, (16,)).astype(jnp.float32)
    return _ffm_sc(i_rows, lin_table.reshape(V), pt, bias16)


# speedup vs baseline: 3.1409x; 1.2686x over previous
import jax
import jax.numpy as jnp
from jax import lax
from jax.experimental import pallas as pl
from jax.experimental.pallas import tpu as pltpu
from jax.experimental.pallas import tpu_sc as plsc

F = 26
V = 100000
D = 32
B = 4096
FIELD_DIM = 3846

NC = 2
NS = 16
NW = NC * NS
SPW = B // NW
NSLOT = 4

NPAIR = (F * (F - 1)) // 2
NQ = (F * D + 127) // 128
NROW = F * NQ
IDX_W = 224
FFM_OFF = 32
CHUNKS = [(FFM_OFF, 128), (FFM_OFF + 128, NROW - 128)]

VB = 2048
VPAD = 100352
NVB = VPAD // VB

_GDN = lax.GatherDimensionNumbers(
    offset_dims=(), collapsed_slice_dims=(0,), start_index_map=(0,))


def _permute(v, idx):
    return lax.gather(v, idx[:, None], _GDN, (1,),
                      mode=lax.GatherScatterMode.PROMISE_IN_BOUNDS)


def _hsum(v, lanes):
    for sh in (8, 4, 2, 1):
        v = v + _permute(v, lanes ^ sh)
    return v


def _tr_body(t_ref, o_ref):
    x = t_ref[...]
    x = x.reshape(128, VB)
    o_ref[...] = jnp.transpose(x)


@jax.jit
def _relayout_tc(tt):
    return pl.pallas_call(
        _tr_body,
        grid=(NQ, NVB),
        in_specs=[pl.BlockSpec((4, D, VB), lambda q, k: (q, 0, k))],
        out_specs=pl.BlockSpec((VB, 128), lambda q, k: (q * NVB + k, 0)),
        out_shape=jax.ShapeDtypeStruct((NQ * VPAD, 128), jnp.float32),
    )(tt)


def _sc_body(i_hbm, lin_hbm, pt_hbm, bias_hbm, out_hbm,
             idx_v, g_v, l_v, outb_v, bias_v, sems, isems):
    wid = lax.axis_index("s") * NC + lax.axis_index("c")
    base = wid * SPW

    pltpu.sync_copy(bias_hbm, bias_v)

    lanes = lax.iota(jnp.int32, 16)
    zero_f = jnp.zeros((16,), jnp.float32)
    bias_vec = bias_v[...]
    for s in range(NSLOT):
        l_v[s][pl.ds(16, 16)] = zero_f

    def idx_start(g, s):
        pltpu.async_copy(i_hbm.at[base + g], idx_v[s], isems[s])

    def idx_wait(s):
        pltpu.make_async_copy(i_hbm.at[base], idx_v[s], isems[s]).wait()

    def issue(s):
        pltpu.async_copy(lin_hbm.at[idx_v[s].at[pl.ds(0, F)]],
                         l_v[s].at[pl.ds(0, F)], sems[s])
        for off, n in CHUNKS:
            pltpu.async_copy(pt_hbm.at[idx_v[s].at[pl.ds(off, n)]],
                             g_v[s].at[pl.ds(off - FFM_OFF, n)], sems[s])

    def drain(s):
        pltpu.make_async_copy(lin_hbm.at[idx_v[s].at[pl.ds(0, F)]],
                              l_v[s].at[pl.ds(0, F)], sems[s]).wait()
        for off, n in CHUNKS:
            pltpu.make_async_copy(pt_hbm.at[idx_v[s].at[pl.ds(off, n)]],
                                  g_v[s].at[pl.ds(off - FFM_OFF, n)],
                                  sems[s]).wait()

    def compute(s):
        gs = g_v[s]

        def pbody(_, carry):
            i, j, a0, a1 = carry
            ra = i * NQ + (j >> 2)
            ca = (j & 3) * 32
            rb = j * NQ + (i >> 2)
            cb = (i & 3) * 32
            a0 = a0 + gs[ra, pl.ds(ca, 16)] * gs[rb, pl.ds(cb, 16)]
            a1 = a1 + gs[ra, pl.ds(ca + 16, 16)] * gs[rb, pl.ds(cb + 16, 16)]
            j2 = j + 1
            wrap = j2 == F
            i2 = jnp.where(wrap, i + 1, i)
            j3 = jnp.where(wrap, i + 2, j2)
            return i2, j3, a0, a1

        _, _, a0, a1 = lax.fori_loop(
            0, NPAIR, pbody,
            (jnp.int32(0), jnp.int32(1), zero_f, zero_f), unroll=13)
        lin = l_v[s][pl.ds(0, 16)] + l_v[s][pl.ds(16, 16)]
        return _hsum(a0 + a1 + lin, lanes) + bias_vec

    for s in range(NSLOT):
        idx_start(s, s)
    for s in range(2):
        idx_wait(s)
        issue(s)

    def lbody(t, res):
        g0 = t * NSLOT
        for s in range(NSLOT):
            g = g0 + s
            drain(s)

            @pl.when(g + 2 < SPW)
            def _():
                idx_wait((s + 2) % NSLOT)
                issue((s + 2) % NSLOT)

            @pl.when(g + NSLOT < SPW)
            def _():
                idx_start(g + NSLOT, s)

            res = jnp.where(lanes == g % 16, compute(s), res)

            @pl.when(g % 16 == 15)
            def _():
                outb_v[pl.ds(g - 15, 16)] = res
        return res

    lax.fori_loop(0, SPW // NSLOT, lbody, zero_f)

    for k in range(SPW // 16):
        v = outb_v[pl.ds(k * 16, 16)]
        outb_v[pl.ds(k * 16, 16)] = 1.0 / (1.0 + jnp.exp(-v))
    pltpu.sync_copy(outb_v, out_hbm.at[pl.ds(base, SPW)])


@jax.jit
def _ffm_sc(i_rows, lin_table, pt, bias16):
    mesh = plsc.VectorSubcoreMesh(core_axis_name="c", subcore_axis_name="s")
    run = pl.kernel(
        _sc_body,
        out_type=jax.ShapeDtypeStruct((B,), jnp.float32),
        mesh=mesh,
        compiler_params=pltpu.CompilerParams(use_tc_tiling_on_sc=False),
        scratch_types=[
            [pltpu.VMEM((IDX_W,), jnp.int32) for _ in range(NSLOT)],
            [pltpu.VMEM((NROW, 128), jnp.float32) for _ in range(NSLOT)],
            [pltpu.VMEM((32,), jnp.float32) for _ in range(NSLOT)],
            pltpu.VMEM((SPW,), jnp.float32),
            pltpu.VMEM((16,), jnp.float32),
            [pltpu.SemaphoreType.DMA for _ in range(NSLOT)],
            [pltpu.SemaphoreType.DMA for _ in range(NSLOT)],
        ],
    )
    return run(i_rows, lin_table, pt, bias16)


def kernel(x, offsets, lin_table, lin_bias, ffm_tables):
    idx = x + offsets[None, :]
    tt = jnp.transpose(ffm_tables, (0, 2, 1))
    pt = _relayout_tc(tt)
    ffm_idx = (idx[:, :, None]
               + (jnp.arange(NQ, dtype=jnp.int32) * VPAD)[None, None, :])
    i_rows = jnp.concatenate(
        [idx, jnp.zeros((B, FFM_OFF - F), jnp.int32),
         ffm_idx.reshape(B, NROW),
         jnp.zeros((B, IDX_W - FFM_OFF - NROW), jnp.int32)],
        axis=1)
    bias16 = jnp.broadcast_to(lin_bias, (16,)).astype(jnp.float32)
    return _ffm_sc(i_rows, lin_table.reshape(V), pt, bias16)


# speedup vs baseline: 3.7170x; 1.1834x over previous
import jax
import jax.numpy as jnp
from jax import lax
from jax.experimental import pallas as pl
from jax.experimental.pallas import tpu as pltpu
from jax.experimental.pallas import tpu_sc as plsc

F = 26
V = 100000
D = 32
B = 4096
FIELD_DIM = 3846

NC = 2
NS = 16
NW = NC * NS
SPW = B // NW
NSLOT = 4

NPAIR = (F * (F - 1)) // 2
NQ = (F * D + 127) // 128
NROW = F * NQ
IDX_W = 224
FFM_OFF = 32
CHUNKS = [(FFM_OFF, 128), (FFM_OFF + 128, NROW - 128)]

VB = 4096
VPAD = 102400
NVB = VPAD // VB

_GDN = lax.GatherDimensionNumbers(
    offset_dims=(), collapsed_slice_dims=(0,), start_index_map=(0,))


def _permute(v, idx):
    return lax.gather(v, idx[:, None], _GDN, (1,),
                      mode=lax.GatherScatterMode.PROMISE_IN_BOUNDS)


def _hsum(v, lanes):
    for sh in (8, 4, 2, 1):
        v = v + _permute(v, lanes ^ sh)
    return v


def _tr_body(t_ref, o_ref):
    x = t_ref[...]
    x = x.reshape(128, VB)
    o_ref[...] = jnp.transpose(x)


@jax.jit
def _relayout_tc(tt):
    return pl.pallas_call(
        _tr_body,
        grid=(NQ, NVB),
        in_specs=[pl.BlockSpec((4, D, VB), lambda q, k: (q, 0, k))],
        out_specs=pl.BlockSpec((VB, 128), lambda q, k: (q * NVB + k, 0)),
        out_shape=jax.ShapeDtypeStruct((NQ * VPAD, 128), jnp.float32),
    )(tt)


def _sc_body(i_hbm, lin_hbm, pt_hbm, bias_hbm, out_hbm,
             idx_v, g_v, l_v, outb_v, bias_v, sems, isems):
    wid = lax.axis_index("s") * NC + lax.axis_index("c")
    base = wid * SPW

    pltpu.sync_copy(bias_hbm, bias_v)

    lanes = lax.iota(jnp.int32, 16)
    zero_f = jnp.zeros((16,), jnp.float32)
    bias_vec = bias_v[...]
    for s in range(NSLOT):
        l_v[s][pl.ds(16, 16)] = zero_f

    def idx_start(g, s):
        pltpu.async_copy(i_hbm.at[base + g], idx_v[s], isems[s])

    def idx_wait(s):
        pltpu.make_async_copy(i_hbm.at[base], idx_v[s], isems[s]).wait()

    def issue(s):
        pltpu.async_copy(lin_hbm.at[idx_v[s].at[pl.ds(0, F)]],
                         l_v[s].at[pl.ds(0, F)], sems[s])
        for off, n in CHUNKS:
            pltpu.async_copy(pt_hbm.at[idx_v[s].at[pl.ds(off, n)]],
                             g_v[s].at[pl.ds(off - FFM_OFF, n)], sems[s])

    def drain(s):
        pltpu.make_async_copy(lin_hbm.at[idx_v[s].at[pl.ds(0, F)]],
                              l_v[s].at[pl.ds(0, F)], sems[s]).wait()
        for off, n in CHUNKS:
            pltpu.make_async_copy(pt_hbm.at[idx_v[s].at[pl.ds(off, n)]],
                                  g_v[s].at[pl.ds(off - FFM_OFF, n)],
                                  sems[s]).wait()

    def compute(s):
        gs = g_v[s]

        def pbody(_, carry):
            i, j, a0, a1 = carry
            ra = i * NQ + (j >> 2)
            ca = (j & 3) * 32
            rb = j * NQ + (i >> 2)
            cb = (i & 3) * 32
            a0 = a0 + gs[ra, pl.ds(ca, 16)] * gs[rb, pl.ds(cb, 16)]
            a1 = a1 + gs[ra, pl.ds(ca + 16, 16)] * gs[rb, pl.ds(cb + 16, 16)]
            j2 = j + 1
            wrap = j2 == F
            i2 = jnp.where(wrap, i + 1, i)
            j3 = jnp.where(wrap, i + 2, j2)
            return i2, j3, a0, a1

        _, _, a0, a1 = lax.fori_loop(
            0, NPAIR, pbody,
            (jnp.int32(0), jnp.int32(1), zero_f, zero_f), unroll=13)
        lin = l_v[s][pl.ds(0, 16)] + l_v[s][pl.ds(16, 16)]
        return _hsum(a0 + a1 + lin, lanes) + bias_vec

    for s in range(NSLOT):
        idx_start(s, s)
    for s in range(2):
        idx_wait(s)
        issue(s)

    def lbody(t, res):
        g0 = t * NSLOT
        for s in range(NSLOT):
            g = g0 + s
            drain(s)

            @pl.when(g + 2 < SPW)
            def _():
                idx_wait((s + 2) % NSLOT)
                issue((s + 2) % NSLOT)

            @pl.when(g + NSLOT < SPW)
            def _():
                idx_start(g + NSLOT, s)

            res = jnp.where(lanes == g % 16, compute(s), res)

            @pl.when(g % 16 == 15)
            def _():
                outb_v[pl.ds(g - 15, 16)] = res
        return res

    lax.fori_loop(0, SPW // NSLOT, lbody, zero_f)

    for k in range(SPW // 16):
        v = outb_v[pl.ds(k * 16, 16)]
        outb_v[pl.ds(k * 16, 16)] = 1.0 / (1.0 + jnp.exp(-v))
    pltpu.sync_copy(outb_v, out_hbm.at[pl.ds(base, SPW)])


@jax.jit
def _ffm_sc(i_rows, lin_table, pt, bias16):
    mesh = plsc.VectorSubcoreMesh(core_axis_name="c", subcore_axis_name="s")
    run = pl.kernel(
        _sc_body,
        out_type=jax.ShapeDtypeStruct((B,), jnp.float32),
        mesh=mesh,
        compiler_params=pltpu.CompilerParams(use_tc_tiling_on_sc=False),
        scratch_types=[
            [pltpu.VMEM((IDX_W,), jnp.int32) for _ in range(NSLOT)],
            [pltpu.VMEM((NROW, 128), jnp.float32) for _ in range(NSLOT)],
            [pltpu.VMEM((32,), jnp.float32) for _ in range(NSLOT)],
            pltpu.VMEM((SPW,), jnp.float32),
            pltpu.VMEM((16,), jnp.float32),
            [pltpu.SemaphoreType.DMA for _ in range(NSLOT)],
            [pltpu.SemaphoreType.DMA for _ in range(NSLOT)],
        ],
    )
    return run(i_rows, lin_table, pt, bias16)


def kernel(x, offsets, lin_table, lin_bias, ffm_tables):
    idx = x + offsets[None, :]
    tt = jnp.transpose(ffm_tables, (0, 2, 1))
    pt = _relayout_tc(tt)
    ffm_idx = (idx[:, :, None]
               + (jnp.arange(NQ, dtype=jnp.int32) * VPAD)[None, None, :])
    i_rows = jnp.concatenate(
        [idx, jnp.zeros((B, FFM_OFF - F), jnp.int32),
         ffm_idx.reshape(B, NROW),
         jnp.zeros((B, IDX_W - FFM_OFF - NROW), jnp.int32)],
        axis=1)
    bias16 = jnp.broadcast_to(lin_bias, (16,)).astype(jnp.float32)
    return _ffm_sc(i_rows, lin_table.reshape(V), pt, bias16)


# speedup vs baseline: 3.9798x; 1.0707x over previous
import jax
import jax.numpy as jnp
from jax import lax
from jax.experimental import pallas as pl
from jax.experimental.pallas import tpu as pltpu
from jax.experimental.pallas import tpu_sc as plsc

F = 26
V = 100000
D = 32
B = 4096
FIELD_DIM = 3846

NC = 2
NS = 16
NW = NC * NS
SPW = B // NW
NSLOT = 4

NPAIR = (F * (F - 1)) // 2
NQ = (F * D + 127) // 128
NROW = F * NQ
IDX_W = 224
FFM_OFF = 32
CHUNKS = [(FFM_OFF, 128), (FFM_OFF + 128, NROW - 128)]

VB = 8192
VPAD = 106496
NVB = VPAD // VB

_GDN = lax.GatherDimensionNumbers(
    offset_dims=(), collapsed_slice_dims=(0,), start_index_map=(0,))


def _permute(v, idx):
    return lax.gather(v, idx[:, None], _GDN, (1,),
                      mode=lax.GatherScatterMode.PROMISE_IN_BOUNDS)


def _hsum(v, lanes):
    for sh in (8, 4, 2, 1):
        v = v + _permute(v, lanes ^ sh)
    return v


def _tr_body(t_ref, o_ref):
    x = t_ref[...]
    x = x.reshape(128, VB)
    o_ref[...] = jnp.transpose(x)


@jax.jit
def _relayout_tc(tt):
    return pl.pallas_call(
        _tr_body,
        grid=(NQ, NVB),
        in_specs=[pl.BlockSpec((4, D, VB), lambda q, k: (q, 0, k))],
        out_specs=pl.BlockSpec((VB, 128), lambda q, k: (q * NVB + k, 0)),
        out_shape=jax.ShapeDtypeStruct((NQ * VPAD, 128), jnp.float32),
    )(tt)


def _sc_body(i_hbm, lin_hbm, pt_hbm, bias_hbm, out_hbm,
             idx_v, g_v, l_v, outb_v, bias_v, sems, isems):
    wid = lax.axis_index("s") * NC + lax.axis_index("c")
    base = wid * SPW

    pltpu.sync_copy(bias_hbm, bias_v)

    lanes = lax.iota(jnp.int32, 16)
    zero_f = jnp.zeros((16,), jnp.float32)
    bias_vec = bias_v[...]
    for s in range(NSLOT):
        l_v[s][pl.ds(16, 16)] = zero_f

    def idx_start(g, s):
        pltpu.async_copy(i_hbm.at[base + g], idx_v[s], isems[s])

    def idx_wait(s):
        pltpu.make_async_copy(i_hbm.at[base], idx_v[s], isems[s]).wait()

    def issue(s):
        pltpu.async_copy(lin_hbm.at[idx_v[s].at[pl.ds(0, F)]],
                         l_v[s].at[pl.ds(0, F)], sems[s])
        for off, n in CHUNKS:
            pltpu.async_copy(pt_hbm.at[idx_v[s].at[pl.ds(off, n)]],
                             g_v[s].at[pl.ds(off - FFM_OFF, n)], sems[s])

    def drain(s):
        pltpu.make_async_copy(lin_hbm.at[idx_v[s].at[pl.ds(0, F)]],
                              l_v[s].at[pl.ds(0, F)], sems[s]).wait()
        for off, n in CHUNKS:
            pltpu.make_async_copy(pt_hbm.at[idx_v[s].at[pl.ds(off, n)]],
                                  g_v[s].at[pl.ds(off - FFM_OFF, n)],
                                  sems[s]).wait()

    def compute(s):
        gs = g_v[s]

        def pbody(_, carry):
            i, j, a0, a1 = carry
            ra = i * NQ + (j >> 2)
            ca = (j & 3) * 32
            rb = j * NQ + (i >> 2)
            cb = (i & 3) * 32
            a0 = a0 + gs[ra, pl.ds(ca, 16)] * gs[rb, pl.ds(cb, 16)]
            a1 = a1 + gs[ra, pl.ds(ca + 16, 16)] * gs[rb, pl.ds(cb + 16, 16)]
            j2 = j + 1
            wrap = j2 == F
            i2 = jnp.where(wrap, i + 1, i)
            j3 = jnp.where(wrap, i + 2, j2)
            return i2, j3, a0, a1

        _, _, a0, a1 = lax.fori_loop(
            0, NPAIR, pbody,
            (jnp.int32(0), jnp.int32(1), zero_f, zero_f), unroll=13)
        lin = l_v[s][pl.ds(0, 16)] + l_v[s][pl.ds(16, 16)]
        return _hsum(a0 + a1 + lin, lanes) + bias_vec

    for s in range(NSLOT):
        idx_start(s, s)
    for s in range(2):
        idx_wait(s)
        issue(s)

    def lbody(t, res):
        g0 = t * NSLOT
        for s in range(NSLOT):
            g = g0 + s
            drain(s)

            @pl.when(g + 2 < SPW)
            def _():
                idx_wait((s + 2) % NSLOT)
                issue((s + 2) % NSLOT)

            @pl.when(g + NSLOT < SPW)
            def _():
                idx_start(g + NSLOT, s)

            res = jnp.where(lanes == g % 16, compute(s), res)

            @pl.when(g % 16 == 15)
            def _():
                outb_v[pl.ds(g - 15, 16)] = res
        return res

    lax.fori_loop(0, SPW // NSLOT, lbody, zero_f)

    for k in range(SPW // 16):
        v = outb_v[pl.ds(k * 16, 16)]
        outb_v[pl.ds(k * 16, 16)] = 1.0 / (1.0 + jnp.exp(-v))
    pltpu.sync_copy(outb_v, out_hbm.at[pl.ds(base, SPW)])


@jax.jit
def _ffm_sc(i_rows, lin_table, pt, bias16):
    mesh = plsc.VectorSubcoreMesh(core_axis_name="c", subcore_axis_name="s")
    run = pl.kernel(
        _sc_body,
        out_type=jax.ShapeDtypeStruct((B,), jnp.float32),
        mesh=mesh,
        compiler_params=pltpu.CompilerParams(use_tc_tiling_on_sc=False),
        scratch_types=[
            [pltpu.VMEM((IDX_W,), jnp.int32) for _ in range(NSLOT)],
            [pltpu.VMEM((NROW, 128), jnp.float32) for _ in range(NSLOT)],
            [pltpu.VMEM((32,), jnp.float32) for _ in range(NSLOT)],
            pltpu.VMEM((SPW,), jnp.float32),
            pltpu.VMEM((16,), jnp.float32),
            [pltpu.SemaphoreType.DMA for _ in range(NSLOT)],
            [pltpu.SemaphoreType.DMA for _ in range(NSLOT)],
        ],
    )
    return run(i_rows, lin_table, pt, bias16)


def kernel(x, offsets, lin_table, lin_bias, ffm_tables):
    idx = x + offsets[None, :]
    tt = jnp.transpose(ffm_tables, (0, 2, 1))
    pt = _relayout_tc(tt)
    ffm_idx = (idx[:, :, None]
               + (jnp.arange(NQ, dtype=jnp.int32) * VPAD)[None, None, :])
    i_rows = jnp.concatenate(
        [idx, jnp.zeros((B, FFM_OFF - F), jnp.int32),
         ffm_idx.reshape(B, NROW),
         jnp.zeros((B, IDX_W - FFM_OFF - NROW), jnp.int32)],
        axis=1)
    bias16 = jnp.broadcast_to(lin_bias, (16,)).astype(jnp.float32)
    return _ffm_sc(i_rows, lin_table.reshape(V), pt, bias16)
